# Initial kernel scaffold; baseline (speedup 1.0000x reference)
import jax
import jax.numpy as jnp
from jax.experimental import pallas as pl


def kernel(x, edge_index, edge_attr, W1, We1, b1, W2, We2, b2):
    raise NotImplementedError("write your pallas kernel here")


# speedup vs baseline: 13.4652x; 13.4652x over previous
import functools

import jax
import jax.numpy as jnp
from jax import lax
from jax.experimental import pallas as pl
from jax.experimental.pallas import tpu as pltpu
from jax.experimental.pallas import tpu_sc as plsc

N = 10000
E = 320000
D = 128
DH = D // 2
DE = 16
DDG = 8

NC = 2
NS = 16
NW = NC * NS
CH = 125
EPW = E // NW
NCHUNK_P = EPW // CH
EPT = E // NS
NCHUNK_E = EPT // CH
NPAD = 10240
RPT = NPAD // NS
ZR = 320


@functools.lru_cache(maxsize=None)
def _mesh():
    return plsc.VectorSubcoreMesh(
        core_axis_name="c", subcore_axis_name="s",
        num_cores=NC, num_subcores=NS)


def _leaky(v):
    return jnp.where(v >= 0, v, 0.01 * v)


def _pre_body(ea_hbm, dst_hbm, ones_hbm, zero_hbm, zero8_hbm, out_ea, out_dg,
              dsti, ea_buf, ones_buf, st_buf, st8_buf, acc_ea, acc_dg):
    c = lax.axis_index("c")
    s = lax.axis_index("s")
    wid = c * NS + s

    pltpu.sync_copy(zero_hbm, st_buf)
    pltpu.sync_copy(st_buf, acc_ea.at[pl.ds(s * RPT, RPT)])
    pltpu.sync_copy(zero8_hbm, st8_buf)
    pltpu.sync_copy(st8_buf, acc_dg.at[pl.ds(s * RPT, RPT)])
    pltpu.sync_copy(ones_hbm, ones_buf)
    pltpu.sync_copy(dst_hbm.at[wid], dsti)
    plsc.subcore_barrier()

    def body(j, carry):
        pltpu.sync_copy(ea_hbm.at[wid, j], ea_buf)
        pltpu.sync_copy(ea_buf, acc_ea.at[dsti.at[j]], add=True)
        pltpu.sync_copy(ones_buf, acc_dg.at[dsti.at[j]], add=True)
        return carry

    lax.fori_loop(0, NCHUNK_P, body, 0)
    plsc.subcore_barrier()

    pltpu.sync_copy(acc_ea.at[pl.ds(s * RPT, RPT)], st_buf)
    pltpu.sync_copy(st_buf, out_ea.at[c, pl.ds(s * RPT, RPT)])
    pltpu.sync_copy(acc_dg.at[pl.ds(s * RPT, RPT)], st8_buf)
    pltpu.sync_copy(st8_buf, out_dg.at[c, pl.ds(s * RPT, RPT)])


@functools.lru_cache(maxsize=None)
def _precompute():
    return pl.kernel(
        _pre_body,
        out_type=(jax.ShapeDtypeStruct((NC, NPAD, DE), jnp.float32),
                  jax.ShapeDtypeStruct((NC, NPAD, DDG), jnp.float32)),
        mesh=_mesh(),
        scratch_types=[
            pltpu.VMEM((NCHUNK_P, CH), jnp.int32),
            pltpu.VMEM((CH, DE), jnp.float32),
            pltpu.VMEM((CH, DDG), jnp.float32),
            pltpu.VMEM((RPT, DE), jnp.float32),
            pltpu.VMEM((RPT, DDG), jnp.float32),
            pltpu.VMEM_SHARED((NPAD, DE), jnp.float32),
            pltpu.VMEM_SHARED((NPAD, DDG), jnp.float32),
        ],
        compiler_params=pltpu.CompilerParams(use_tc_tiling_on_sc=False),
    )


def _edge_body(t_hbm, src2_hbm, dst_hbm, zero_hbm, out_p,
               srci, dsti, rows0, rows1, st_buf, acc, sem0, sem1):
    c = lax.axis_index("c")
    s = lax.axis_index("s")

    pltpu.sync_copy(zero_hbm, st_buf)
    pltpu.sync_copy(st_buf, acc.at[pl.ds(s * RPT, ZR)])
    pltpu.sync_copy(st_buf, acc.at[pl.ds(s * RPT + ZR, ZR)])
    pltpu.sync_copy(src2_hbm.at[c, s], srci)
    pltpu.sync_copy(dst_hbm.at[s], dsti)
    plsc.subcore_barrier()

    pltpu.async_copy(t_hbm.at[srci.at[0]], rows0, sem0)
    pltpu.async_copy(t_hbm.at[srci.at[1]], rows1, sem1)

    def body(i, carry):
        for b, (rows, sem) in enumerate(((rows0, sem0), (rows1, sem1))):
            jj = 2 * i + b
            pltpu.make_async_copy(t_hbm.at[srci.at[jj]], rows, sem).wait()
            pltpu.sync_copy(rows, acc.at[dsti.at[jj]], add=True)

            @pl.when(jj + 2 < NCHUNK_E)
            def _():
                pltpu.async_copy(t_hbm.at[srci.at[jj + 2]], rows, sem)
        return carry

    lax.fori_loop(0, NCHUNK_E // 2, body, 0)
    plsc.subcore_barrier()

    pltpu.sync_copy(acc.at[pl.ds(s * RPT, ZR)], st_buf)
    pltpu.sync_copy(st_buf, out_p.at[c, pl.ds(s * RPT, ZR)])
    pltpu.sync_copy(acc.at[pl.ds(s * RPT + ZR, ZR)], st_buf)
    pltpu.sync_copy(st_buf, out_p.at[c, pl.ds(s * RPT + ZR, ZR)])


@functools.lru_cache(maxsize=None)
def _edge_pass():
    return pl.kernel(
        _edge_body,
        out_type=jax.ShapeDtypeStruct((NC, NPAD, DH), jnp.float32),
        mesh=_mesh(),
        scratch_types=[
            pltpu.VMEM((NCHUNK_E, CH), jnp.int32),
            pltpu.VMEM((NCHUNK_E, CH), jnp.int32),
            pltpu.VMEM((CH, DH), jnp.float32),
            pltpu.VMEM((CH, DH), jnp.float32),
            pltpu.VMEM((ZR, DH), jnp.float32),
            pltpu.VMEM_SHARED((NPAD, DH), jnp.float32),
            pltpu.SemaphoreType.DMA,
            pltpu.SemaphoreType.DMA,
        ],
        compiler_params=pltpu.CompilerParams(use_tc_tiling_on_sc=False),
    )


_BLK = 1000
_GRID = N // _BLK


def _r_from_dg(dg_blk):
    deg = dg_blk[0, :, 0:1] + dg_blk[1, :, 0:1]
    return lax.rsqrt(jnp.maximum(deg, 1.0))


def _split_halves(t_ref, prod):
    t_ref[0] = prod[:, :DH]
    t_ref[1] = prod[:, DH:]


def _tc_pre_body(x_ref, w_ref, we_ref, b_ref, ea_ref, dg_ref, t_ref, base_ref):
    r = _r_from_dg(dg_ref)
    prod = jnp.dot(x_ref[...], w_ref[...],
                   preferred_element_type=jnp.float32) * r
    _split_halves(t_ref, prod)
    ea = ea_ref[0] + ea_ref[1]
    base_ref[...] = jnp.dot(ea, we_ref[...],
                            preferred_element_type=jnp.float32) + b_ref[...]


def _tc_mid_body(p_ref, base_ref, w_ref, we_ref, b_ref, ea_ref, dg_ref,
                 t_ref, base2_ref):
    r = _r_from_dg(dg_ref)
    agg = jnp.concatenate([p_ref[0], p_ref[1]], axis=1)
    h = _leaky(r * agg + base_ref[...])
    prod = jnp.dot(h, w_ref[...], preferred_element_type=jnp.float32) * r
    _split_halves(t_ref, prod)
    ea = ea_ref[0] + ea_ref[1]
    base2_ref[...] = jnp.dot(ea, we_ref[...],
                             preferred_element_type=jnp.float32) + b_ref[...]


def _tc_post_body(p_ref, base_ref, dg_ref, out_ref):
    r = _r_from_dg(dg_ref)
    agg = jnp.concatenate([p_ref[0], p_ref[1]], axis=1)
    out_ref[...] = _leaky(r * agg + base_ref[...])


def _row_spec(width):
    return pl.BlockSpec((_BLK, width), lambda i: (i, 0))


def _pair_spec(width):
    return pl.BlockSpec((2, _BLK, width), lambda i: (0, i, 0))


def _full_spec(a, b):
    return pl.BlockSpec((a, b), lambda i: (0, 0))


_f32 = jnp.float32
_sds = jax.ShapeDtypeStruct

_tc_pre = pl.pallas_call(
    _tc_pre_body,
    grid=(_GRID,),
    in_specs=[_row_spec(D), _full_spec(D, D), _full_spec(DE, D),
              _full_spec(1, D), _pair_spec(DE), _pair_spec(DDG)],
    out_specs=(_pair_spec(DH), _row_spec(D)),
    out_shape=(_sds((NC, N, DH), _f32), _sds((N, D), _f32)),
)

_tc_mid = pl.pallas_call(
    _tc_mid_body,
    grid=(_GRID,),
    in_specs=[_pair_spec(DH), _row_spec(D), _full_spec(D, D),
              _full_spec(DE, D), _full_spec(1, D), _pair_spec(DE),
              _pair_spec(DDG)],
    out_specs=(_pair_spec(DH), _row_spec(D)),
    out_shape=(_sds((NC, N, DH), _f32), _sds((N, D), _f32)),
)

_tc_post = pl.pallas_call(
    _tc_post_body,
    grid=(_GRID,),
    in_specs=[_pair_spec(DH), _row_spec(D), _pair_spec(DDG)],
    out_specs=_row_spec(D),
    out_shape=_sds((N, D), _f32),
)


def kernel(x, edge_index, edge_attr, W1, We1, b1, W2, We2, b2):
    src = edge_index[0]
    dst = edge_index[1]
    dstp = dst.reshape(NW, NCHUNK_P, CH)
    srce = src.reshape(NS, NCHUNK_E, CH)
    src2 = jnp.stack([srce, srce + N])
    dste = dst.reshape(NS, NCHUNK_E, CH)
    ea = edge_attr.reshape(NW, NCHUNK_P, CH, DE)
    ones_e = jnp.ones((CH, DDG), _f32)
    z_de = jnp.zeros((RPT, DE), _f32)
    z_8 = jnp.zeros((RPT, DDG), _f32)
    z_dh = jnp.zeros((ZR, DH), _f32)

    pea, pdg = _precompute()(ea, dstp, ones_e, z_de, z_8)
    pea, pdg = pea[:, :N], pdg[:, :N]

    t1, base1 = _tc_pre(x, W1, We1, b1.reshape(1, D), pea, pdg)
    p1 = _edge_pass()(t1.reshape(NC * N, DH), src2, dste, z_dh)[:, :N]
    t2, base2 = _tc_mid(p1, base1, W2, We2, b2.reshape(1, D), pea, pdg)
    p2 = _edge_pass()(t2.reshape(NC * N, DH), src2, dste, z_dh)[:, :N]
    return _tc_post(p2, base2, pdg)


# speedup vs baseline: 14.2040x; 1.0549x over previous
import functools

import jax
import jax.numpy as jnp
from jax import lax
from jax.experimental import pallas as pl
from jax.experimental.pallas import tpu as pltpu
from jax.experimental.pallas import tpu_sc as plsc

N = 10000
E = 320000
D = 128
DH = D // 2
DE = 16
DDG = 8

NC = 2
NS = 16
NW = NC * NS
CH = 80
EPW = E // NW
NCHUNK_P = EPW // CH
EPT = E // NS
NCHUNK_E = EPT // CH
NPAD = 10240
RPT = NPAD // NS
ZR = 320
TAIL = N - (NS - 1) * RPT - ZR
EA_C = 32


@functools.lru_cache(maxsize=None)
def _mesh():
    return plsc.VectorSubcoreMesh(
        core_axis_name="c", subcore_axis_name="s",
        num_cores=NC, num_subcores=NS)


def _leaky(v):
    return jnp.where(v >= 0, v, 0.01 * v)


def _pre_body(ea_hbm, dst_hbm, ones_hbm, z16_hbm, z8_hbm, out_ag,
              dsti, ea_buf, ones_buf, st16, st8, acc_ea, acc_dg):
    c = lax.axis_index("c")
    s = lax.axis_index("s")
    wid = c * NS + s
    base = wid * EPW

    pltpu.sync_copy(z16_hbm, st16)
    pltpu.sync_copy(st16, acc_ea.at[pl.ds(s * RPT, RPT)])
    pltpu.sync_copy(z8_hbm, st8)
    pltpu.sync_copy(st8, acc_dg.at[pl.ds(s * RPT, RPT)])
    pltpu.sync_copy(ones_hbm, ones_buf)
    pltpu.sync_copy(dst_hbm.at[pl.ds(base, EPW)], dsti)
    plsc.subcore_barrier()

    def body(j, carry):
        idx = dsti.at[pl.ds(j * CH, CH)]
        pltpu.sync_copy(ea_hbm.at[pl.ds(base + j * CH, CH)], ea_buf)
        pltpu.sync_copy(ea_buf, acc_ea.at[idx], add=True)
        pltpu.sync_copy(ones_buf, acc_dg.at[idx], add=True)
        return carry

    lax.fori_loop(0, NCHUNK_P, body, 0)
    plsc.subcore_barrier()

    nvalid = N - (NS - 1) * RPT
    pltpu.sync_copy(acc_ea.at[pl.ds(s * RPT, RPT)], st16)
    pltpu.sync_copy(acc_dg.at[pl.ds(s * RPT, RPT)], st8)

    @pl.when(s < NS - 1)
    def _():
        pltpu.sync_copy(
            st16, out_ag.at[pl.ds(s * RPT, RPT), pl.ds(c * DE, DE)])
        pltpu.sync_copy(
            st8, out_ag.at[pl.ds(s * RPT, RPT), pl.ds(EA_C + c * DDG, DDG)])

    @pl.when(s == NS - 1)
    def _():
        pltpu.sync_copy(
            st16.at[pl.ds(0, nvalid)],
            out_ag.at[pl.ds((NS - 1) * RPT, nvalid), pl.ds(c * DE, DE)])
        pltpu.sync_copy(
            st8.at[pl.ds(0, nvalid)],
            out_ag.at[pl.ds((NS - 1) * RPT, nvalid),
                      pl.ds(EA_C + c * DDG, DDG)])


@functools.lru_cache(maxsize=None)
def _precompute():
    return pl.kernel(
        _pre_body,
        out_type=jax.ShapeDtypeStruct((N, D), jnp.float32),
        mesh=_mesh(),
        scratch_types=[
            pltpu.VMEM((EPW,), jnp.int32),
            pltpu.VMEM((CH, DE), jnp.float32),
            pltpu.VMEM((CH, DDG), jnp.float32),
            pltpu.VMEM((RPT, DE), jnp.float32),
            pltpu.VMEM((RPT, DDG), jnp.float32),
            pltpu.VMEM_SHARED((NPAD, DE), jnp.float32),
            pltpu.VMEM_SHARED((NPAD, DDG), jnp.float32),
        ],
        compiler_params=pltpu.CompilerParams(use_tc_tiling_on_sc=False),
    )


def _edge_body(t_hbm, src2_hbm, dst_hbm, zero_hbm, out_p,
               srci, dsti, rows0, rows1, st_buf, acc, sem0, sem1):
    c = lax.axis_index("c")
    s = lax.axis_index("s")

    pltpu.sync_copy(zero_hbm, st_buf)
    pltpu.sync_copy(st_buf, acc.at[pl.ds(s * RPT, ZR)])
    pltpu.sync_copy(st_buf, acc.at[pl.ds(s * RPT + ZR, ZR)])
    pltpu.sync_copy(src2_hbm.at[pl.ds(c * E + s * EPT, EPT)], srci)
    pltpu.sync_copy(dst_hbm.at[pl.ds(s * EPT, EPT)], dsti)
    plsc.subcore_barrier()

    pltpu.async_copy(t_hbm.at[srci.at[pl.ds(0, CH)]], rows0, sem0)
    pltpu.async_copy(t_hbm.at[srci.at[pl.ds(CH, CH)]], rows1, sem1)

    def body(i, carry):
        for b, (rows, sem) in enumerate(((rows0, sem0), (rows1, sem1))):
            jj = 2 * i + b
            pltpu.make_async_copy(
                t_hbm.at[srci.at[pl.ds(jj * CH, CH)]], rows, sem).wait()
            pltpu.sync_copy(rows, acc.at[dsti.at[pl.ds(jj * CH, CH)]],
                            add=True)

            @pl.when(jj + 2 < NCHUNK_E)
            def _():
                pltpu.async_copy(
                    t_hbm.at[srci.at[pl.ds((jj + 2) * CH, CH)]], rows, sem)
        return carry

    lax.fori_loop(0, NCHUNK_E // 2, body, 0)
    plsc.subcore_barrier()

    pltpu.sync_copy(acc.at[pl.ds(s * RPT, ZR)], st_buf)
    pltpu.sync_copy(
        st_buf, out_p.at[pl.ds(s * RPT, ZR), pl.ds(c * DH, DH)])
    pltpu.sync_copy(acc.at[pl.ds(s * RPT + ZR, ZR)], st_buf)

    @pl.when(s < NS - 1)
    def _():
        pltpu.sync_copy(
            st_buf, out_p.at[pl.ds(s * RPT + ZR, ZR), pl.ds(c * DH, DH)])

    @pl.when(s == NS - 1)
    def _():
        pltpu.sync_copy(
            st_buf.at[pl.ds(0, TAIL)],
            out_p.at[pl.ds((NS - 1) * RPT + ZR, TAIL), pl.ds(c * DH, DH)])


@functools.lru_cache(maxsize=None)
def _edge_pass():
    return pl.kernel(
        _edge_body,
        out_type=jax.ShapeDtypeStruct((N, D), jnp.float32),
        mesh=_mesh(),
        scratch_types=[
            pltpu.VMEM((EPT,), jnp.int32),
            pltpu.VMEM((EPT,), jnp.int32),
            pltpu.VMEM((CH, DH), jnp.float32),
            pltpu.VMEM((CH, DH), jnp.float32),
            pltpu.VMEM((ZR, DH), jnp.float32),
            pltpu.VMEM_SHARED((NPAD, DH), jnp.float32),
            pltpu.SemaphoreType.DMA,
            pltpu.SemaphoreType.DMA,
        ],
        compiler_params=pltpu.CompilerParams(use_tc_tiling_on_sc=False),
    )


_BLK = 1000
_GRID = N // _BLK


def _r_ea_from_ag(ag):
    ea = ag[:, 0:DE] + ag[:, DE:2 * DE]
    deg = ag[:, EA_C:EA_C + 1] + ag[:, EA_C + DDG:EA_C + DDG + 1]
    return lax.rsqrt(jnp.maximum(deg, 1.0)), ea


def _tc_pre_body(x_ref, w_ref, we_ref, b_ref, ag_ref, t_ref, base_ref):
    r, ea = _r_ea_from_ag(ag_ref[...])
    t_ref[...] = jnp.dot(x_ref[...], w_ref[...],
                         preferred_element_type=jnp.float32) * r
    base_ref[...] = jnp.dot(ea, we_ref[...],
                            preferred_element_type=jnp.float32) + b_ref[...]


def _tc_mid_body(p_ref, base_ref, w_ref, we_ref, b_ref, ag_ref,
                 t_ref, base2_ref):
    r, ea = _r_ea_from_ag(ag_ref[...])
    h = _leaky(r * p_ref[...] + base_ref[...])
    t_ref[...] = jnp.dot(h, w_ref[...], preferred_element_type=jnp.float32) * r
    base2_ref[...] = jnp.dot(ea, we_ref[...],
                             preferred_element_type=jnp.float32) + b_ref[...]


def _tc_post_body(p_ref, base_ref, ag_ref, out_ref):
    r, _ = _r_ea_from_ag(ag_ref[...])
    out_ref[...] = _leaky(r * p_ref[...] + base_ref[...])


def _row_spec(width):
    return pl.BlockSpec((_BLK, width), lambda i: (i, 0))


def _full_spec(a, b):
    return pl.BlockSpec((a, b), lambda i: (0, 0))


_f32 = jnp.float32
_sds = jax.ShapeDtypeStruct

_tc_pre = pl.pallas_call(
    _tc_pre_body,
    grid=(_GRID,),
    in_specs=[_row_spec(D), _full_spec(D, D), _full_spec(DE, D),
              _full_spec(1, D), _row_spec(D)],
    out_specs=(_row_spec(D), _row_spec(D)),
    out_shape=(_sds((N, D), _f32), _sds((N, D), _f32)),
)

_tc_mid = pl.pallas_call(
    _tc_mid_body,
    grid=(_GRID,),
    in_specs=[_row_spec(D), _row_spec(D), _full_spec(D, D),
              _full_spec(DE, D), _full_spec(1, D), _row_spec(D)],
    out_specs=(_row_spec(D), _row_spec(D)),
    out_shape=(_sds((N, D), _f32), _sds((N, D), _f32)),
)

_tc_post = pl.pallas_call(
    _tc_post_body,
    grid=(_GRID,),
    in_specs=[_row_spec(D), _row_spec(D), _row_spec(D)],
    out_specs=_row_spec(D),
    out_shape=_sds((N, D), _f32),
)


def kernel(x, edge_index, edge_attr, W1, We1, b1, W2, We2, b2):
    src = edge_index[0]
    dst = edge_index[1]
    src2 = jnp.concatenate([2 * src, 2 * src + 1])
    ones8 = jnp.ones((CH, DDG), _f32)
    z16 = jnp.zeros((RPT, DE), _f32)
    z8 = jnp.zeros((RPT, DDG), _f32)
    z_dh = jnp.zeros((ZR, DH), _f32)
    b1r = b1.reshape(1, D)
    b2r = b2.reshape(1, D)

    ag = _precompute()(edge_attr, dst, ones8, z16, z8)

    t1, base1 = _tc_pre(x, W1, We1, b1r, ag)
    p1 = _edge_pass()(t1.reshape(2 * N, DH), src2, dst, z_dh)
    t2, base2 = _tc_mid(p1, base1, W2, We2, b2r, ag)
    p2 = _edge_pass()(t2.reshape(2 * N, DH), src2, dst, z_dh)
    return _tc_post(p2, base2, ag)


# speedup vs baseline: 18.3846x; 1.2943x over previous
import functools

import jax
import jax.numpy as jnp
from jax import lax
from jax.experimental import pallas as pl
from jax.experimental.pallas import tpu as pltpu
from jax.experimental.pallas import tpu_sc as plsc

N = 10000
E = 320000
D = 128
DH = D // 2
DE = 16
DDG = 8

NC = 2
NS = 16
NW = NC * NS
CH = 128
EPW = E // NW
NFULL_P = EPW // CH
TAIL_P = EPW - NFULL_P * CH
EPT = E // NS
NFULL_E = EPT // CH
TAIL_E = EPT - NFULL_E * CH
NPAD = 10240
RPT = NPAD // NS
ZR = 320
TAIL = N - (NS - 1) * RPT - ZR
NV = N - (NS - 1) * RPT


@functools.lru_cache(maxsize=None)
def _mesh():
    return plsc.VectorSubcoreMesh(
        core_axis_name="c", subcore_axis_name="s",
        num_cores=NC, num_subcores=NS)


def _leaky(v):
    return jnp.where(v >= 0, v, 0.01 * v)


def _deg_body(dst_hbm, ones_hbm, z8_hbm, out_dg,
              dsti, ones_buf, st8, acc_dg, sem0, sem1):
    c = lax.axis_index("c")
    s = lax.axis_index("s")
    base = (c * NS + s) * EPW

    pltpu.sync_copy(z8_hbm, st8)
    pltpu.sync_copy(st8, acc_dg.at[pl.ds(s * RPT, RPT)])
    pltpu.sync_copy(ones_hbm, ones_buf)
    pltpu.sync_copy(dst_hbm.at[pl.ds(base, EPW)], dsti)
    plsc.subcore_barrier()

    def body(i, carry):
        for b, sem in enumerate((sem0, sem1)):
            jj = 2 * i + b
            pltpu.async_copy(
                ones_buf, acc_dg.at[dsti.at[pl.ds(jj * CH, CH)]], sem,
                add=True)
        for b, sem in enumerate((sem0, sem1)):
            jj = 2 * i + b
            pltpu.make_async_copy(
                ones_buf, acc_dg.at[dsti.at[pl.ds(jj * CH, CH)]], sem).wait()
        return carry

    lax.fori_loop(0, NFULL_P // 2, body, 0)
    pltpu.sync_copy(ones_buf.at[pl.ds(0, TAIL_P)],
                    acc_dg.at[dsti.at[pl.ds(NFULL_P * CH, TAIL_P)]], add=True)
    plsc.subcore_barrier()

    pltpu.sync_copy(acc_dg.at[pl.ds(s * RPT, RPT)], st8)

    @pl.when(s < NS - 1)
    def _():
        pltpu.sync_copy(
            st8, out_dg.at[pl.ds(s * RPT, RPT), pl.ds(c * DDG, DDG)])

    @pl.when(s == NS - 1)
    def _():
        pltpu.sync_copy(
            st8.at[pl.ds(0, NV)],
            out_dg.at[pl.ds((NS - 1) * RPT, NV), pl.ds(c * DDG, DDG)])


@functools.lru_cache(maxsize=None)
def _deg_pass():
    return pl.kernel(
        _deg_body,
        out_type=jax.ShapeDtypeStruct((N, 2 * DDG), jnp.float32),
        mesh=_mesh(),
        scratch_types=[
            pltpu.VMEM((EPW,), jnp.int32),
            pltpu.VMEM((CH, DDG), jnp.float32),
            pltpu.VMEM((RPT, DDG), jnp.float32),
            pltpu.VMEM_SHARED((NPAD, DDG), jnp.float32),
            pltpu.SemaphoreType.DMA,
            pltpu.SemaphoreType.DMA,
        ],
        compiler_params=pltpu.CompilerParams(use_tc_tiling_on_sc=False),
    )


def _ea_body(ea_hbm, dst_hbm, z16_hbm, out_ea,
             dsti, ea_buf0, ea_buf1, st16, acc_ea, sem0, sem1):
    c = lax.axis_index("c")
    s = lax.axis_index("s")
    base = (c * NS + s) * EPW

    pltpu.sync_copy(z16_hbm, st16)
    pltpu.sync_copy(st16, acc_ea.at[pl.ds(s * RPT, RPT)])
    pltpu.sync_copy(dst_hbm.at[pl.ds(base, EPW)], dsti)
    plsc.subcore_barrier()

    pltpu.async_copy(ea_hbm.at[pl.ds(base, CH)], ea_buf0, sem0)
    pltpu.async_copy(ea_hbm.at[pl.ds(base + CH, CH)], ea_buf1, sem1)

    def body(i, carry):
        for b, (buf, sem) in enumerate(((ea_buf0, sem0), (ea_buf1, sem1))):
            jj = 2 * i + b
            pltpu.make_async_copy(
                ea_hbm.at[pl.ds(base + jj * CH, CH)], buf, sem).wait()
            pltpu.sync_copy(buf, acc_ea.at[dsti.at[pl.ds(jj * CH, CH)]],
                            add=True)

            @pl.when(jj + 2 < NFULL_P)
            def _():
                pltpu.async_copy(
                    ea_hbm.at[pl.ds(base + (jj + 2) * CH, CH)], buf, sem)
        return carry

    lax.fori_loop(0, NFULL_P // 2, body, 0)

    pltpu.sync_copy(ea_hbm.at[pl.ds(base + NFULL_P * CH, TAIL_P)],
                    ea_buf0.at[pl.ds(0, TAIL_P)])
    pltpu.sync_copy(ea_buf0.at[pl.ds(0, TAIL_P)],
                    acc_ea.at[dsti.at[pl.ds(NFULL_P * CH, TAIL_P)]], add=True)
    plsc.subcore_barrier()

    pltpu.sync_copy(acc_ea.at[pl.ds(s * RPT, RPT)], st16)

    @pl.when(s < NS - 1)
    def _():
        pltpu.sync_copy(
            st16, out_ea.at[pl.ds(s * RPT, RPT), pl.ds(c * DE, DE)])

    @pl.when(s == NS - 1)
    def _():
        pltpu.sync_copy(
            st16.at[pl.ds(0, NV)],
            out_ea.at[pl.ds((NS - 1) * RPT, NV), pl.ds(c * DE, DE)])


@functools.lru_cache(maxsize=None)
def _ea_pass():
    return pl.kernel(
        _ea_body,
        out_type=jax.ShapeDtypeStruct((N, 2 * DE), jnp.float32),
        mesh=_mesh(),
        scratch_types=[
            pltpu.VMEM((EPW,), jnp.int32),
            pltpu.VMEM((CH, DE), jnp.float32),
            pltpu.VMEM((CH, DE), jnp.float32),
            pltpu.VMEM((RPT, DE), jnp.float32),
            pltpu.VMEM_SHARED((NPAD, DE), jnp.float32),
            pltpu.SemaphoreType.DMA,
            pltpu.SemaphoreType.DMA,
        ],
        compiler_params=pltpu.CompilerParams(use_tc_tiling_on_sc=False),
    )


def _edge_body(t_hbm, src2_hbm, dst_hbm, zero_hbm, out_p,
               srci, dsti, rows0, rows1, st_buf, acc, sem0, sem1):
    c = lax.axis_index("c")
    s = lax.axis_index("s")

    pltpu.sync_copy(zero_hbm, st_buf)
    pltpu.sync_copy(st_buf, acc.at[pl.ds(s * RPT, ZR)])
    pltpu.sync_copy(st_buf, acc.at[pl.ds(s * RPT + ZR, ZR)])
    pltpu.sync_copy(src2_hbm.at[pl.ds(c * E + s * EPT, EPT)], srci)
    pltpu.sync_copy(dst_hbm.at[pl.ds(s * EPT, EPT)], dsti)
    plsc.subcore_barrier()

    pltpu.async_copy(t_hbm.at[srci.at[pl.ds(0, CH)]], rows0, sem0)
    pltpu.async_copy(t_hbm.at[srci.at[pl.ds(CH, CH)]], rows1, sem1)

    def body(i, carry):
        for b, (rows, sem) in enumerate(((rows0, sem0), (rows1, sem1))):
            jj = 2 * i + b
            pltpu.make_async_copy(
                t_hbm.at[srci.at[pl.ds(jj * CH, CH)]], rows, sem).wait()
            pltpu.sync_copy(rows, acc.at[dsti.at[pl.ds(jj * CH, CH)]],
                            add=True)

            @pl.when(jj + 2 < NFULL_E)
            def _():
                pltpu.async_copy(
                    t_hbm.at[srci.at[pl.ds((jj + 2) * CH, CH)]], rows, sem)
        return carry

    lax.fori_loop(0, NFULL_E // 2, body, 0)

    pltpu.async_copy(
        t_hbm.at[srci.at[pl.ds(NFULL_E * CH, TAIL_E)]],
        rows0.at[pl.ds(0, TAIL_E)], sem0)
    pltpu.make_async_copy(
        t_hbm.at[srci.at[pl.ds(NFULL_E * CH, TAIL_E)]],
        rows0.at[pl.ds(0, TAIL_E)], sem0).wait()
    pltpu.sync_copy(rows0.at[pl.ds(0, TAIL_E)],
                    acc.at[dsti.at[pl.ds(NFULL_E * CH, TAIL_E)]], add=True)
    plsc.subcore_barrier()

    pltpu.sync_copy(acc.at[pl.ds(s * RPT, ZR)], st_buf)
    pltpu.sync_copy(
        st_buf, out_p.at[pl.ds(s * RPT, ZR), pl.ds(c * DH, DH)])
    pltpu.sync_copy(acc.at[pl.ds(s * RPT + ZR, ZR)], st_buf)

    @pl.when(s < NS - 1)
    def _():
        pltpu.sync_copy(
            st_buf, out_p.at[pl.ds(s * RPT + ZR, ZR), pl.ds(c * DH, DH)])

    @pl.when(s == NS - 1)
    def _():
        pltpu.sync_copy(
            st_buf.at[pl.ds(0, TAIL)],
            out_p.at[pl.ds((NS - 1) * RPT + ZR, TAIL), pl.ds(c * DH, DH)])


@functools.lru_cache(maxsize=None)
def _edge_pass():
    return pl.kernel(
        _edge_body,
        out_type=jax.ShapeDtypeStruct((N, D), jnp.float32),
        mesh=_mesh(),
        scratch_types=[
            pltpu.VMEM((EPT,), jnp.int32),
            pltpu.VMEM((EPT,), jnp.int32),
            pltpu.VMEM((CH, DH), jnp.float32),
            pltpu.VMEM((CH, DH), jnp.float32),
            pltpu.VMEM((ZR, DH), jnp.float32),
            pltpu.VMEM_SHARED((NPAD, DH), jnp.float32),
            pltpu.SemaphoreType.DMA,
            pltpu.SemaphoreType.DMA,
        ],
        compiler_params=pltpu.CompilerParams(use_tc_tiling_on_sc=False),
    )


_BLK = 1000
_GRID = N // _BLK


def _r_from_dg(dg):
    deg = dg[:, 0:1] + dg[:, DDG:DDG + 1]
    return lax.rsqrt(jnp.maximum(deg, 1.0))


def _tc_pre_body(x_ref, w_ref, dg_ref, t_ref):
    r = _r_from_dg(dg_ref[...])
    t_ref[...] = jnp.dot(x_ref[...], w_ref[...],
                         preferred_element_type=jnp.float32) * r


def _tc_mid_body(p_ref, ea_ref, dg_ref, w_ref, we1_ref, b1_ref, we2_ref,
                 b2_ref, t_ref, base2_ref):
    r = _r_from_dg(dg_ref[...])
    ea = ea_ref[:, 0:DE] + ea_ref[:, DE:2 * DE]
    base1 = jnp.dot(ea, we1_ref[...],
                    preferred_element_type=jnp.float32) + b1_ref[...]
    h = _leaky(r * p_ref[...] + base1)
    t_ref[...] = jnp.dot(h, w_ref[...], preferred_element_type=jnp.float32) * r
    base2_ref[...] = jnp.dot(ea, we2_ref[...],
                             preferred_element_type=jnp.float32) + b2_ref[...]


def _tc_post_body(p_ref, base_ref, dg_ref, out_ref):
    r = _r_from_dg(dg_ref[...])
    out_ref[...] = _leaky(r * p_ref[...] + base_ref[...])


def _row_spec(width):
    return pl.BlockSpec((_BLK, width), lambda i: (i, 0))


def _full_spec(a, b):
    return pl.BlockSpec((a, b), lambda i: (0, 0))


_f32 = jnp.float32
_sds = jax.ShapeDtypeStruct

_tc_pre = pl.pallas_call(
    _tc_pre_body,
    grid=(_GRID,),
    in_specs=[_row_spec(D), _full_spec(D, D), _row_spec(2 * DDG)],
    out_specs=_row_spec(D),
    out_shape=_sds((N, D), _f32),
)

_tc_mid = pl.pallas_call(
    _tc_mid_body,
    grid=(_GRID,),
    in_specs=[_row_spec(D), _row_spec(2 * DE), _row_spec(2 * DDG),
              _full_spec(D, D), _full_spec(DE, D), _full_spec(1, D),
              _full_spec(DE, D), _full_spec(1, D)],
    out_specs=(_row_spec(D), _row_spec(D)),
    out_shape=(_sds((N, D), _f32), _sds((N, D), _f32)),
)

_tc_post = pl.pallas_call(
    _tc_post_body,
    grid=(_GRID,),
    in_specs=[_row_spec(D), _row_spec(D), _row_spec(2 * DDG)],
    out_specs=_row_spec(D),
    out_shape=_sds((N, D), _f32),
)


def kernel(x, edge_index, edge_attr, W1, We1, b1, W2, We2, b2):
    src = edge_index[0]
    dst = edge_index[1]
    src2 = jnp.concatenate([2 * src, 2 * src + 1])
    ones8 = jnp.ones((CH, DDG), _f32)
    z16 = jnp.zeros((RPT, DE), _f32)
    z8 = jnp.zeros((RPT, DDG), _f32)
    z_dh = jnp.zeros((ZR, DH), _f32)

    dg = _deg_pass()(dst, ones8, z8)
    t1 = _tc_pre(x, W1, dg)
    p1 = _edge_pass()(t1.reshape(2 * N, DH), src2, dst, z_dh)
    ea = _ea_pass()(edge_attr, dst, z16)
    t2, base2 = _tc_mid(p1, ea, dg, W2, We1, b1.reshape(1, D),
                        We2, b2.reshape(1, D))
    p2 = _edge_pass()(t2.reshape(2 * N, DH), src2, dst, z_dh)
    return _tc_post(p2, base2, dg)


# speedup vs baseline: 20.3140x; 1.1049x over previous
import functools

import jax
import jax.numpy as jnp
from jax import lax
from jax.experimental import pallas as pl
from jax.experimental.pallas import tpu as pltpu
from jax.experimental.pallas import tpu_sc as plsc

N = 10000
E = 320000
D = 128
DH = D // 2
DE = 16
DDG = 8

NC = 2
NS = 16
NW = NC * NS
CH = 128
EPW = E // NW
NFULL_P = EPW // CH
TAIL_P = EPW - NFULL_P * CH
EPT = E // NS
NFULL_E = EPT // CH
TAIL_E = EPT - NFULL_E * CH
NPAD = 10240
RPT = NPAD // NS
ZR = 320
TAIL = N - (NS - 1) * RPT - ZR
NV = N - (NS - 1) * RPT


@functools.lru_cache(maxsize=None)
def _mesh():
    return plsc.VectorSubcoreMesh(
        core_axis_name="c", subcore_axis_name="s",
        num_cores=NC, num_subcores=NS)


def _leaky(v):
    return jnp.where(v >= 0, v, 0.01 * v)


def _deg_body(dst_hbm, ones_hbm, z8_hbm, out_dg,
              dsti, ones_buf, st8, acc_dg, sem0, sem1):
    c = lax.axis_index("c")
    s = lax.axis_index("s")
    base = (c * NS + s) * EPW

    pltpu.sync_copy(z8_hbm, st8)
    pltpu.sync_copy(st8, acc_dg.at[pl.ds(s * RPT, RPT)])
    pltpu.sync_copy(ones_hbm, ones_buf)
    pltpu.sync_copy(dst_hbm.at[pl.ds(base, EPW)], dsti)
    plsc.subcore_barrier()

    def body(i, carry):
        for b, sem in enumerate((sem0, sem1)):
            jj = 2 * i + b
            pltpu.async_copy(
                ones_buf, acc_dg.at[dsti.at[pl.ds(jj * CH, CH)]], sem,
                add=True)
        for b, sem in enumerate((sem0, sem1)):
            jj = 2 * i + b
            pltpu.make_async_copy(
                ones_buf, acc_dg.at[dsti.at[pl.ds(jj * CH, CH)]], sem).wait()
        return carry

    lax.fori_loop(0, NFULL_P // 2, body, 0)
    pltpu.sync_copy(ones_buf.at[pl.ds(0, TAIL_P)],
                    acc_dg.at[dsti.at[pl.ds(NFULL_P * CH, TAIL_P)]], add=True)
    plsc.subcore_barrier()

    pltpu.sync_copy(acc_dg.at[pl.ds(s * RPT, RPT)], st8)

    @pl.when(s < NS - 1)
    def _():
        pltpu.sync_copy(
            st8, out_dg.at[pl.ds(s * RPT, RPT), pl.ds(c * DDG, DDG)])

    @pl.when(s == NS - 1)
    def _():
        pltpu.sync_copy(
            st8.at[pl.ds(0, NV)],
            out_dg.at[pl.ds((NS - 1) * RPT, NV), pl.ds(c * DDG, DDG)])


@functools.lru_cache(maxsize=None)
def _deg_pass():
    return pl.kernel(
        _deg_body,
        out_type=jax.ShapeDtypeStruct((N, 2 * DDG), jnp.float32),
        mesh=_mesh(),
        scratch_types=[
            pltpu.VMEM((EPW,), jnp.int32),
            pltpu.VMEM((CH, DDG), jnp.float32),
            pltpu.VMEM((RPT, DDG), jnp.float32),
            pltpu.VMEM_SHARED((NPAD, DDG), jnp.float32),
            pltpu.SemaphoreType.DMA,
            pltpu.SemaphoreType.DMA,
        ],
        compiler_params=pltpu.CompilerParams(use_tc_tiling_on_sc=False),
    )


def _ea_body(ea_hbm, dst_hbm, z16_hbm, dep_hbm, out_ea,
             dsti, ea_buf0, ea_buf1, st16, acc_ea, sem0, sem1):
    del dep_hbm
    c = lax.axis_index("c")
    s = lax.axis_index("s")
    base = (c * NS + s) * EPW

    pltpu.sync_copy(z16_hbm, st16)
    pltpu.sync_copy(st16, acc_ea.at[pl.ds(s * RPT, RPT)])
    pltpu.sync_copy(dst_hbm.at[pl.ds(base, EPW)], dsti)
    plsc.subcore_barrier()

    pltpu.async_copy(ea_hbm.at[pl.ds(base, CH)], ea_buf0, sem0)
    pltpu.async_copy(ea_hbm.at[pl.ds(base + CH, CH)], ea_buf1, sem1)

    def body(i, carry):
        for b, (buf, sem) in enumerate(((ea_buf0, sem0), (ea_buf1, sem1))):
            jj = 2 * i + b
            pltpu.make_async_copy(
                ea_hbm.at[pl.ds(base + jj * CH, CH)], buf, sem).wait()
            pltpu.sync_copy(buf, acc_ea.at[dsti.at[pl.ds(jj * CH, CH)]],
                            add=True)

            @pl.when(jj + 2 < NFULL_P)
            def _():
                pltpu.async_copy(
                    ea_hbm.at[pl.ds(base + (jj + 2) * CH, CH)], buf, sem)
        return carry

    lax.fori_loop(0, NFULL_P // 2, body, 0)

    pltpu.sync_copy(ea_hbm.at[pl.ds(base + NFULL_P * CH, TAIL_P)],
                    ea_buf0.at[pl.ds(0, TAIL_P)])
    pltpu.sync_copy(ea_buf0.at[pl.ds(0, TAIL_P)],
                    acc_ea.at[dsti.at[pl.ds(NFULL_P * CH, TAIL_P)]], add=True)
    plsc.subcore_barrier()

    pltpu.sync_copy(acc_ea.at[pl.ds(s * RPT, RPT)], st16)

    @pl.when(s < NS - 1)
    def _():
        pltpu.sync_copy(
            st16, out_ea.at[pl.ds(s * RPT, RPT), pl.ds(c * DE, DE)])

    @pl.when(s == NS - 1)
    def _():
        pltpu.sync_copy(
            st16.at[pl.ds(0, NV)],
            out_ea.at[pl.ds((NS - 1) * RPT, NV), pl.ds(c * DE, DE)])


@functools.lru_cache(maxsize=None)
def _ea_pass():
    return pl.kernel(
        _ea_body,
        out_type=jax.ShapeDtypeStruct((N, 2 * DE), jnp.float32),
        mesh=_mesh(),
        scratch_types=[
            pltpu.VMEM((EPW,), jnp.int32),
            pltpu.VMEM((CH, DE), jnp.float32),
            pltpu.VMEM((CH, DE), jnp.float32),
            pltpu.VMEM((RPT, DE), jnp.float32),
            pltpu.VMEM_SHARED((NPAD, DE), jnp.float32),
            pltpu.SemaphoreType.DMA,
            pltpu.SemaphoreType.DMA,
        ],
        compiler_params=pltpu.CompilerParams(use_tc_tiling_on_sc=False),
    )


def _edge_body(t_hbm, src2_hbm, dst_hbm, zero_hbm, out_p,
               srci, dsti, rows0, rows1, st_buf, acc, sem0, sem1):
    c = lax.axis_index("c")
    s = lax.axis_index("s")

    pltpu.sync_copy(zero_hbm, st_buf)
    pltpu.sync_copy(st_buf, acc.at[pl.ds(s * RPT, ZR)])
    pltpu.sync_copy(st_buf, acc.at[pl.ds(s * RPT + ZR, ZR)])
    pltpu.sync_copy(src2_hbm.at[pl.ds(c * E + s * EPT, EPT)], srci)
    pltpu.sync_copy(dst_hbm.at[pl.ds(s * EPT, EPT)], dsti)
    plsc.subcore_barrier()

    pltpu.async_copy(t_hbm.at[srci.at[pl.ds(0, CH)]], rows0, sem0)
    pltpu.async_copy(t_hbm.at[srci.at[pl.ds(CH, CH)]], rows1, sem1)

    def body(i, carry):
        for b, (rows, sem) in enumerate(((rows0, sem0), (rows1, sem1))):
            jj = 2 * i + b
            pltpu.make_async_copy(
                t_hbm.at[srci.at[pl.ds(jj * CH, CH)]], rows, sem).wait()
            pltpu.sync_copy(rows, acc.at[dsti.at[pl.ds(jj * CH, CH)]],
                            add=True)

            @pl.when(jj + 2 < NFULL_E)
            def _():
                pltpu.async_copy(
                    t_hbm.at[srci.at[pl.ds((jj + 2) * CH, CH)]], rows, sem)
        return carry

    lax.fori_loop(0, NFULL_E // 2, body, 0)

    pltpu.async_copy(
        t_hbm.at[srci.at[pl.ds(NFULL_E * CH, TAIL_E)]],
        rows0.at[pl.ds(0, TAIL_E)], sem0)
    pltpu.make_async_copy(
        t_hbm.at[srci.at[pl.ds(NFULL_E * CH, TAIL_E)]],
        rows0.at[pl.ds(0, TAIL_E)], sem0).wait()
    pltpu.sync_copy(rows0.at[pl.ds(0, TAIL_E)],
                    acc.at[dsti.at[pl.ds(NFULL_E * CH, TAIL_E)]], add=True)
    plsc.subcore_barrier()

    pltpu.sync_copy(acc.at[pl.ds(s * RPT, ZR)], st_buf)
    pltpu.sync_copy(
        st_buf, out_p.at[pl.ds(s * RPT, ZR), pl.ds(c * DH, DH)])
    pltpu.sync_copy(acc.at[pl.ds(s * RPT + ZR, ZR)], st_buf)

    @pl.when(s < NS - 1)
    def _():
        pltpu.sync_copy(
            st_buf, out_p.at[pl.ds(s * RPT + ZR, ZR), pl.ds(c * DH, DH)])

    @pl.when(s == NS - 1)
    def _():
        pltpu.sync_copy(
            st_buf.at[pl.ds(0, TAIL)],
            out_p.at[pl.ds((NS - 1) * RPT + ZR, TAIL), pl.ds(c * DH, DH)])


@functools.lru_cache(maxsize=None)
def _edge_pass():
    return pl.kernel(
        _edge_body,
        out_type=jax.ShapeDtypeStruct((N, D), jnp.float32),
        mesh=_mesh(),
        scratch_types=[
            pltpu.VMEM((EPT,), jnp.int32),
            pltpu.VMEM((EPT,), jnp.int32),
            pltpu.VMEM((CH, DH), jnp.float32),
            pltpu.VMEM((CH, DH), jnp.float32),
            pltpu.VMEM((ZR, DH), jnp.float32),
            pltpu.VMEM_SHARED((NPAD, DH), jnp.float32),
            pltpu.SemaphoreType.DMA,
            pltpu.SemaphoreType.DMA,
        ],
        compiler_params=pltpu.CompilerParams(use_tc_tiling_on_sc=False),
    )


_BLK = 1000
_GRID = N // _BLK


def _r_from_dg(dg):
    deg = dg[:, 0:1] + dg[:, DDG:DDG + 1]
    return lax.rsqrt(jnp.maximum(deg, 1.0))


def _tc_pre_body(x_ref, w_ref, dg_ref, t_ref):
    r = _r_from_dg(dg_ref[...])
    t_ref[...] = jnp.dot(x_ref[...], w_ref[...],
                         preferred_element_type=jnp.float32) * r


def _tc_mid_body(p_ref, ea_ref, dg_ref, w_ref, we1_ref, b1_ref, we2_ref,
                 b2_ref, t_ref, base2_ref):
    r = _r_from_dg(dg_ref[...])
    ea = ea_ref[:, 0:DE] + ea_ref[:, DE:2 * DE]
    base1 = jnp.dot(ea, we1_ref[...],
                    preferred_element_type=jnp.float32) + b1_ref[...]
    h = _leaky(r * p_ref[...] + base1)
    t_ref[...] = jnp.dot(h, w_ref[...], preferred_element_type=jnp.float32) * r
    base2_ref[...] = jnp.dot(ea, we2_ref[...],
                             preferred_element_type=jnp.float32) + b2_ref[...]


def _tc_post_body(p_ref, base_ref, dg_ref, out_ref):
    r = _r_from_dg(dg_ref[...])
    out_ref[...] = _leaky(r * p_ref[...] + base_ref[...])


def _row_spec(width):
    return pl.BlockSpec((_BLK, width), lambda i: (i, 0))


def _full_spec(a, b):
    return pl.BlockSpec((a, b), lambda i: (0, 0))


_f32 = jnp.float32
_sds = jax.ShapeDtypeStruct

_tc_pre = pl.pallas_call(
    _tc_pre_body,
    grid=(_GRID,),
    in_specs=[_row_spec(D), _full_spec(D, D), _row_spec(2 * DDG)],
    out_specs=_row_spec(D),
    out_shape=_sds((N, D), _f32),
)

_tc_mid = pl.pallas_call(
    _tc_mid_body,
    grid=(_GRID,),
    in_specs=[_row_spec(D), _row_spec(2 * DE), _row_spec(2 * DDG),
              _full_spec(D, D), _full_spec(DE, D), _full_spec(1, D),
              _full_spec(DE, D), _full_spec(1, D)],
    out_specs=(_row_spec(D), _row_spec(D)),
    out_shape=(_sds((N, D), _f32), _sds((N, D), _f32)),
)

_tc_post = pl.pallas_call(
    _tc_post_body,
    grid=(_GRID,),
    in_specs=[_row_spec(D), _row_spec(D), _row_spec(2 * DDG)],
    out_specs=_row_spec(D),
    out_shape=_sds((N, D), _f32),
)


def kernel(x, edge_index, edge_attr, W1, We1, b1, W2, We2, b2):
    src = edge_index[0]
    dst = edge_index[1]
    src2 = jnp.concatenate([2 * src, 2 * src + 1])
    ones8 = jnp.ones((CH, DDG), _f32)
    z16 = jnp.zeros((RPT, DE), _f32)
    z8 = jnp.zeros((RPT, DDG), _f32)
    z_dh = jnp.zeros((ZR, DH), _f32)

    dg = _deg_pass()(dst, ones8, z8)
    t1 = _tc_pre(x, W1, dg)
    p1 = _edge_pass()(t1.reshape(2 * N, DH), src2, dst, z_dh)
    ea = _ea_pass()(edge_attr, dst, z16, p1)
    t2, base2 = _tc_mid(p1, ea, dg, W2, We1, b1.reshape(1, D),
                        We2, b2.reshape(1, D))
    p2 = _edge_pass()(t2.reshape(2 * N, DH), src2, dst, z_dh)
    return _tc_post(p2, base2, dg)


# speedup vs baseline: 20.3296x; 1.0008x over previous
import functools

import jax
import jax.numpy as jnp
from jax import lax
from jax.experimental import pallas as pl
from jax.experimental.pallas import tpu as pltpu
from jax.experimental.pallas import tpu_sc as plsc

N = 10000
E = 320000
D = 128
DH = D // 2
DE = 16
DDG = 8

NC = 2
NS = 16
NW = NC * NS
CH = 128
EPW = E // NW
NFULL_P = EPW // CH
TAIL_P = EPW - NFULL_P * CH
EPT = E // NS
NFULL_E = EPT // CH
TAIL_E = EPT - NFULL_E * CH
NPAD = 10240
RPT = NPAD // NS
ZR = 320
TAIL = N - (NS - 1) * RPT - ZR
NV = N - (NS - 1) * RPT


@functools.lru_cache(maxsize=None)
def _mesh():
    return plsc.VectorSubcoreMesh(
        core_axis_name="c", subcore_axis_name="s",
        num_cores=NC, num_subcores=NS)


def _leaky(v):
    return jnp.where(v >= 0, v, 0.01 * v)


def _deg_body(dst_hbm, ones_hbm, z8_hbm, out_dg,
              dsti, ones_buf, st8, acc_dg, sem0, sem1):
    c = lax.axis_index("c")
    s = lax.axis_index("s")
    base = (c * NS + s) * EPW

    pltpu.sync_copy(z8_hbm, st8)
    pltpu.sync_copy(st8, acc_dg.at[pl.ds(s * RPT, RPT)])
    pltpu.sync_copy(ones_hbm, ones_buf)
    pltpu.sync_copy(dst_hbm.at[pl.ds(base, EPW)], dsti)
    plsc.subcore_barrier()

    def body(i, carry):
        for b, sem in enumerate((sem0, sem1)):
            jj = 2 * i + b
            pltpu.async_copy(
                ones_buf, acc_dg.at[dsti.at[pl.ds(jj * CH, CH)]], sem,
                add=True)
        for b, sem in enumerate((sem0, sem1)):
            jj = 2 * i + b
            pltpu.make_async_copy(
                ones_buf, acc_dg.at[dsti.at[pl.ds(jj * CH, CH)]], sem).wait()
        return carry

    lax.fori_loop(0, NFULL_P // 2, body, 0)
    pltpu.sync_copy(ones_buf.at[pl.ds(0, TAIL_P)],
                    acc_dg.at[dsti.at[pl.ds(NFULL_P * CH, TAIL_P)]], add=True)
    plsc.subcore_barrier()

    pltpu.sync_copy(acc_dg.at[pl.ds(s * RPT, RPT)], st8)

    @pl.when(s < NS - 1)
    def _():
        pltpu.sync_copy(
            st8, out_dg.at[pl.ds(s * RPT, RPT), pl.ds(c * DDG, DDG)])

    @pl.when(s == NS - 1)
    def _():
        pltpu.sync_copy(
            st8.at[pl.ds(0, NV)],
            out_dg.at[pl.ds((NS - 1) * RPT, NV), pl.ds(c * DDG, DDG)])


@functools.lru_cache(maxsize=None)
def _deg_pass():
    return pl.kernel(
        _deg_body,
        out_type=jax.ShapeDtypeStruct((N, 2 * DDG), jnp.float32),
        mesh=_mesh(),
        scratch_types=[
            pltpu.VMEM((EPW,), jnp.int32),
            pltpu.VMEM((CH, DDG), jnp.float32),
            pltpu.VMEM((RPT, DDG), jnp.float32),
            pltpu.VMEM_SHARED((NPAD, DDG), jnp.float32),
            pltpu.SemaphoreType.DMA,
            pltpu.SemaphoreType.DMA,
        ],
        compiler_params=pltpu.CompilerParams(use_tc_tiling_on_sc=False),
    )


def _ea_body(ea_hbm, dst_hbm, z16_hbm, dep_hbm, out_ea,
             dsti, ea_buf0, ea_buf1, st16, acc_ea, sem0, sem1):
    del dep_hbm
    c = lax.axis_index("c")
    s = lax.axis_index("s")
    base = (c * NS + s) * EPW

    pltpu.sync_copy(z16_hbm, st16)
    pltpu.sync_copy(st16, acc_ea.at[pl.ds(s * RPT, RPT)])
    pltpu.sync_copy(dst_hbm.at[pl.ds(base, EPW)], dsti)
    plsc.subcore_barrier()

    pltpu.async_copy(ea_hbm.at[pl.ds(base, CH)], ea_buf0, sem0)
    pltpu.async_copy(ea_hbm.at[pl.ds(base + CH, CH)], ea_buf1, sem1)

    def body(i, carry):
        for b, (buf, sem) in enumerate(((ea_buf0, sem0), (ea_buf1, sem1))):
            jj = 2 * i + b
            pltpu.make_async_copy(
                ea_hbm.at[pl.ds(base + jj * CH, CH)], buf, sem).wait()
            pltpu.sync_copy(buf, acc_ea.at[dsti.at[pl.ds(jj * CH, CH)]],
                            add=True)

            @pl.when(jj + 2 < NFULL_P)
            def _():
                pltpu.async_copy(
                    ea_hbm.at[pl.ds(base + (jj + 2) * CH, CH)], buf, sem)
        return carry

    lax.fori_loop(0, NFULL_P // 2, body, 0)

    pltpu.sync_copy(ea_hbm.at[pl.ds(base + NFULL_P * CH, TAIL_P)],
                    ea_buf0.at[pl.ds(0, TAIL_P)])
    pltpu.sync_copy(ea_buf0.at[pl.ds(0, TAIL_P)],
                    acc_ea.at[dsti.at[pl.ds(NFULL_P * CH, TAIL_P)]], add=True)
    plsc.subcore_barrier()

    pltpu.sync_copy(acc_ea.at[pl.ds(s * RPT, RPT)], st16)

    @pl.when(s < NS - 1)
    def _():
        pltpu.sync_copy(
            st16, out_ea.at[pl.ds(s * RPT, RPT), pl.ds(c * DE, DE)])

    @pl.when(s == NS - 1)
    def _():
        pltpu.sync_copy(
            st16.at[pl.ds(0, NV)],
            out_ea.at[pl.ds((NS - 1) * RPT, NV), pl.ds(c * DE, DE)])


@functools.lru_cache(maxsize=None)
def _ea_pass():
    return pl.kernel(
        _ea_body,
        out_type=jax.ShapeDtypeStruct((N, 2 * DE), jnp.float32),
        mesh=_mesh(),
        scratch_types=[
            pltpu.VMEM((EPW,), jnp.int32),
            pltpu.VMEM((CH, DE), jnp.float32),
            pltpu.VMEM((CH, DE), jnp.float32),
            pltpu.VMEM((RPT, DE), jnp.float32),
            pltpu.VMEM_SHARED((NPAD, DE), jnp.float32),
            pltpu.SemaphoreType.DMA,
            pltpu.SemaphoreType.DMA,
        ],
        compiler_params=pltpu.CompilerParams(use_tc_tiling_on_sc=False),
    )


def _edge_body(t_hbm, src2_hbm, dst_hbm, zero_hbm, out_p,
               srci, dsti, rows0, rows1, st_buf, acc, sem0, sem1):
    c = lax.axis_index("c")
    s = lax.axis_index("s")

    pltpu.sync_copy(zero_hbm, st_buf)
    pltpu.sync_copy(st_buf, acc.at[pl.ds(s * RPT, ZR)])
    pltpu.sync_copy(st_buf, acc.at[pl.ds(s * RPT + ZR, ZR)])
    pltpu.sync_copy(src2_hbm.at[c, pl.ds(s * EPT, EPT)], srci)
    pltpu.sync_copy(dst_hbm.at[pl.ds(s * EPT, EPT)], dsti)
    plsc.subcore_barrier()

    pltpu.async_copy(t_hbm.at[srci.at[pl.ds(0, CH)]], rows0, sem0)
    pltpu.async_copy(t_hbm.at[srci.at[pl.ds(CH, CH)]], rows1, sem1)

    def body(i, carry):
        for b, (rows, sem) in enumerate(((rows0, sem0), (rows1, sem1))):
            jj = 2 * i + b
            pltpu.make_async_copy(
                t_hbm.at[srci.at[pl.ds(jj * CH, CH)]], rows, sem).wait()
            pltpu.sync_copy(rows, acc.at[dsti.at[pl.ds(jj * CH, CH)]],
                            add=True)

            @pl.when(jj + 2 < NFULL_E)
            def _():
                pltpu.async_copy(
                    t_hbm.at[srci.at[pl.ds((jj + 2) * CH, CH)]], rows, sem)
        return carry

    lax.fori_loop(0, NFULL_E // 2, body, 0)

    pltpu.async_copy(
        t_hbm.at[srci.at[pl.ds(NFULL_E * CH, TAIL_E)]],
        rows0.at[pl.ds(0, TAIL_E)], sem0)
    pltpu.make_async_copy(
        t_hbm.at[srci.at[pl.ds(NFULL_E * CH, TAIL_E)]],
        rows0.at[pl.ds(0, TAIL_E)], sem0).wait()
    pltpu.sync_copy(rows0.at[pl.ds(0, TAIL_E)],
                    acc.at[dsti.at[pl.ds(NFULL_E * CH, TAIL_E)]], add=True)
    plsc.subcore_barrier()

    pltpu.sync_copy(acc.at[pl.ds(s * RPT, ZR)], st_buf)
    pltpu.sync_copy(
        st_buf, out_p.at[pl.ds(s * RPT, ZR), pl.ds(c * DH, DH)])
    pltpu.sync_copy(acc.at[pl.ds(s * RPT + ZR, ZR)], st_buf)

    @pl.when(s < NS - 1)
    def _():
        pltpu.sync_copy(
            st_buf, out_p.at[pl.ds(s * RPT + ZR, ZR), pl.ds(c * DH, DH)])

    @pl.when(s == NS - 1)
    def _():
        pltpu.sync_copy(
            st_buf.at[pl.ds(0, TAIL)],
            out_p.at[pl.ds((NS - 1) * RPT + ZR, TAIL), pl.ds(c * DH, DH)])


@functools.lru_cache(maxsize=None)
def _edge_pass():
    return pl.kernel(
        _edge_body,
        out_type=jax.ShapeDtypeStruct((N, D), jnp.float32),
        mesh=_mesh(),
        scratch_types=[
            pltpu.VMEM((EPT,), jnp.int32),
            pltpu.VMEM((EPT,), jnp.int32),
            pltpu.VMEM((CH, DH), jnp.float32),
            pltpu.VMEM((CH, DH), jnp.float32),
            pltpu.VMEM((ZR, DH), jnp.float32),
            pltpu.VMEM_SHARED((NPAD, DH), jnp.float32),
            pltpu.SemaphoreType.DMA,
            pltpu.SemaphoreType.DMA,
        ],
        compiler_params=pltpu.CompilerParams(use_tc_tiling_on_sc=False),
    )


_BLK = 1000
_GRID = N // _BLK


def _r_from_dg(dg):
    deg = dg[:, 0:1] + dg[:, DDG:DDG + 1]
    return lax.rsqrt(jnp.maximum(deg, 1.0))


def _tc_pre_body(x_ref, w_ref, dg_ref, t_ref):
    r = _r_from_dg(dg_ref[...])
    t_ref[...] = jnp.dot(x_ref[...], w_ref[...],
                         preferred_element_type=jnp.float32) * r


def _tc_mid_body(p_ref, ea_ref, dg_ref, w_ref, we1_ref, b1_ref, we2_ref,
                 b2_ref, t_ref, base2_ref):
    r = _r_from_dg(dg_ref[...])
    ea = ea_ref[:, 0:DE] + ea_ref[:, DE:2 * DE]
    base1 = jnp.dot(ea, we1_ref[...],
                    preferred_element_type=jnp.float32) + b1_ref[...]
    h = _leaky(r * p_ref[...] + base1)
    t_ref[...] = jnp.dot(h, w_ref[...], preferred_element_type=jnp.float32) * r
    base2_ref[...] = jnp.dot(ea, we2_ref[...],
                             preferred_element_type=jnp.float32) + b2_ref[...]


def _tc_post_body(p_ref, base_ref, dg_ref, out_ref):
    r = _r_from_dg(dg_ref[...])
    out_ref[...] = _leaky(r * p_ref[...] + base_ref[...])


def _idx_body(ei_ref, out_ref):
    sv = ei_ref[0]
    out_ref[0] = 2 * sv
    out_ref[1] = 2 * sv + 1


_EBLK = E // 10

_tc_idx = pl.pallas_call(
    _idx_body,
    grid=(10,),
    in_specs=[pl.BlockSpec((2, _EBLK), lambda i: (0, i))],
    out_specs=pl.BlockSpec((2, _EBLK), lambda i: (0, i)),
    out_shape=jax.ShapeDtypeStruct((2, E), jnp.int32),
)


def _row_spec(width):
    return pl.BlockSpec((_BLK, width), lambda i: (i, 0))


def _full_spec(a, b):
    return pl.BlockSpec((a, b), lambda i: (0, 0))


_f32 = jnp.float32
_sds = jax.ShapeDtypeStruct

_tc_pre = pl.pallas_call(
    _tc_pre_body,
    grid=(_GRID,),
    in_specs=[_row_spec(D), _full_spec(D, D), _row_spec(2 * DDG)],
    out_specs=_row_spec(D),
    out_shape=_sds((N, D), _f32),
)

_tc_mid = pl.pallas_call(
    _tc_mid_body,
    grid=(_GRID,),
    in_specs=[_row_spec(D), _row_spec(2 * DE), _row_spec(2 * DDG),
              _full_spec(D, D), _full_spec(DE, D), _full_spec(1, D),
              _full_spec(DE, D), _full_spec(1, D)],
    out_specs=(_row_spec(D), _row_spec(D)),
    out_shape=(_sds((N, D), _f32), _sds((N, D), _f32)),
)

_tc_post = pl.pallas_call(
    _tc_post_body,
    grid=(_GRID,),
    in_specs=[_row_spec(D), _row_spec(D), _row_spec(2 * DDG)],
    out_specs=_row_spec(D),
    out_shape=_sds((N, D), _f32),
)


def kernel(x, edge_index, edge_attr, W1, We1, b1, W2, We2, b2):
    dst = edge_index[1]
    src2 = _tc_idx(edge_index)
    ones8 = jnp.ones((CH, DDG), _f32)
    z16 = jnp.zeros((RPT, DE), _f32)
    z8 = jnp.zeros((RPT, DDG), _f32)
    z_dh = jnp.zeros((ZR, DH), _f32)

    dg = _deg_pass()(dst, ones8, z8)
    t1 = _tc_pre(x, W1, dg)
    p1 = _edge_pass()(t1.reshape(2 * N, DH), src2, dst, z_dh)
    ea = _ea_pass()(edge_attr, dst, z16, p1)
    t2, base2 = _tc_mid(p1, ea, dg, W2, We1, b1.reshape(1, D),
                        We2, b2.reshape(1, D))
    p2 = _edge_pass()(t2.reshape(2 * N, DH), src2, dst, z_dh)
    return _tc_post(p2, base2, dg)
